# baseline (device time: 107858 ns/iter reference)
import jax
import jax.numpy as jnp
from jax import lax
from jax.experimental import pallas as pl
from jax.experimental.pallas import tpu as pltpu

N_DEV = 32


def kernel(x, w_mat):
    m, _ = x.shape
    _, n = w_mat.shape
    m_chunk = m // N_DEV
    n_hops = N_DEV - 1

    def body(x_ref, w_ref, out_ref, p_ref, send_buf, recv_buf,
             send_sems, recv_sems):
        my = lax.axis_index("i")
        left = lax.rem(my + N_DEV - 1, N_DEV)
        right = lax.rem(my + 1, N_DEV)

        barrier_sem = pltpu.get_barrier_semaphore()
        for nbr in (left, right):
            pl.semaphore_signal(
                barrier_sem, inc=1,
                device_id=(nbr,), device_id_type=pl.DeviceIdType.MESH,
            )
        pl.semaphore_wait(barrier_sem, 2)

        p_ref[:, :] = jnp.dot(
            x_ref[:, :], w_ref[:, :], preferred_element_type=jnp.float32
        )

        for s in range(n_hops):
            chunk = lax.rem(my + (N_DEV - 1 - s), N_DEV)
            contrib = p_ref[pl.ds(chunk * m_chunk, m_chunk), :]
            if s == 0:
                send_buf[s, :, :] = contrib
            else:
                send_buf[s, :, :] = recv_buf[s - 1, :, :] + contrib
            rdma = pltpu.make_async_remote_copy(
                src_ref=send_buf.at[s],
                dst_ref=recv_buf.at[s],
                send_sem=send_sems.at[s],
                recv_sem=recv_sems.at[s],
                device_id=(right,),
                device_id_type=pl.DeviceIdType.MESH,
            )
            rdma.start()
            rdma.wait()

        y = recv_buf[n_hops - 1, :, :] + p_ref[pl.ds(my * m_chunk, m_chunk), :]
        out_ref[:, :] = y * jax.nn.sigmoid(y)

    return pl.pallas_call(
        body,
        out_shape=jax.ShapeDtypeStruct((m_chunk, n), jnp.float32),
        in_specs=[
            pl.BlockSpec(memory_space=pltpu.VMEM),
            pl.BlockSpec(memory_space=pltpu.VMEM),
        ],
        out_specs=pl.BlockSpec(memory_space=pltpu.VMEM),
        scratch_shapes=[
            pltpu.VMEM((m, n), jnp.float32),
            pltpu.VMEM((n_hops, m_chunk, n), jnp.float32),
            pltpu.VMEM((n_hops, m_chunk, n), jnp.float32),
            pltpu.SemaphoreType.DMA((n_hops,)),
            pltpu.SemaphoreType.DMA((n_hops,)),
        ],
        compiler_params=pltpu.CompilerParams(collective_id=0),
    )(x, w_mat)


# device time: 65178 ns/iter; 1.6548x vs baseline; 1.6548x over previous
import jax
import jax.numpy as jnp
from jax import lax
from jax.experimental import pallas as pl
from jax.experimental.pallas import tpu as pltpu

N_DEV = 32
N_PLANES = 4
PLANE = 8
CYC = [0, 1, 2, 5, 6, 7, 4, 3]
POS = [CYC.index(i) for i in range(PLANE)]
NEXT = [CYC[(POS[i] + 1) % PLANE] for i in range(PLANE)]
PREV = [CYC[(POS[i] - 1) % PLANE] for i in range(PLANE)]
SEND_TBL = [
    [CYC[(POS[i] - 1 - s) % PLANE] for i in range(PLANE)]
    for s in range(PLANE - 1)
]


def _lut(idx, table):
    r = jnp.int32(table[0])
    for i in range(1, len(table)):
        r = jnp.where(idx == i, jnp.int32(table[i]), r)
    return r


def kernel(x, w_mat):
    m, _ = x.shape
    _, n = w_mat.shape
    m_chunk = m // N_DEV
    nh = n // 2
    mp = m // N_PLANES
    mg = m // PLANE

    def body(x_ref, w_ref, out_ref, p_ref, bstage, sA, sB,
             a1_s, a1_r, b1_s, b1_r, a2_s, a2_r, b2_s, b2_r,
             a1_ss, a1_rs, b1_ss, b1_rs, a2_ss, a2_rs, b2_ss, b2_rs):
        my = lax.axis_index("i")
        j = my // PLANE
        k = lax.rem(my, PLANE)

        z_up = lax.rem(my + PLANE, N_DEV)
        z_down = lax.rem(my + (N_DEV - PLANE), N_DEV)
        plane_next = j * PLANE + _lut(k, NEXT)
        plane_prev = j * PLANE + _lut(k, PREV)

        barrier_sem = pltpu.get_barrier_semaphore()
        for nbr in (z_down, plane_prev):
            pl.semaphore_signal(
                barrier_sem, inc=1,
                device_id=(nbr,), device_id_type=pl.DeviceIdType.MESH,
            )

        p_ref[:, :] = jnp.dot(
            x_ref[:, :], w_ref[:, :], preferred_element_type=jnp.float32
        )
        for kp in range(PLANE):
            for jp in range(N_PLANES):
                bstage[pl.ds(kp * mg + jp * m_chunk, m_chunk), :] = (
                    p_ref[pl.ds((jp * PLANE + kp) * m_chunk, m_chunk),
                          pl.ds(nh, nh)]
                )

        pl.semaphore_wait(barrier_sem, 2)

        def a1_start(s):
            b_idx = lax.rem(j + (N_PLANES - 1 - s), N_PLANES)
            val = p_ref[pl.ds(b_idx * mp, mp), pl.ds(0, nh)]
            if s > 0:
                val = val + a1_r[s - 1, :, :]
            a1_s[s, :, :] = val
            r = pltpu.make_async_remote_copy(
                src_ref=a1_s.at[s], dst_ref=a1_r.at[s],
                send_sem=a1_ss.at[s], recv_sem=a1_rs.at[s],
                device_id=(z_up,), device_id_type=pl.DeviceIdType.MESH,
            )
            r.start()
            return r

        def b1_start(s):
            g = _lut(k, SEND_TBL[s])
            val = bstage[pl.ds(g * mg, mg), :]
            if s > 0:
                val = val + b1_r[s - 1, :, :]
            b1_s[s, :, :] = val
            r = pltpu.make_async_remote_copy(
                src_ref=b1_s.at[s], dst_ref=b1_r.at[s],
                send_sem=b1_ss.at[s], recv_sem=b1_rs.at[s],
                device_id=(plane_next,), device_id_type=pl.DeviceIdType.MESH,
            )
            r.start()
            return r

        for t in range(PLANE - 1):
            rb = b1_start(t)
            ra = a1_start(t) if t < N_PLANES - 1 else None
            rb.wait()
            if ra is not None:
                ra.wait()

        sA[:, :] = (
            a1_r[N_PLANES - 2, :, :] + p_ref[pl.ds(j * mp, mp), pl.ds(0, nh)]
        )
        sB[:, :] = b1_r[PLANE - 2, :, :] + bstage[pl.ds(k * mg, mg), :]

        def a2_start(s):
            g = _lut(k, SEND_TBL[s])
            val = sA[pl.ds(g * m_chunk, m_chunk), :]
            if s > 0:
                val = val + a2_r[s - 1, :, :]
            a2_s[s, :, :] = val
            r = pltpu.make_async_remote_copy(
                src_ref=a2_s.at[s], dst_ref=a2_r.at[s],
                send_sem=a2_ss.at[s], recv_sem=a2_rs.at[s],
                device_id=(plane_next,), device_id_type=pl.DeviceIdType.MESH,
            )
            r.start()
            return r

        def b2_start(s):
            b_idx = lax.rem(j + (N_PLANES - 1 - s), N_PLANES)
            val = sB[pl.ds(b_idx * m_chunk, m_chunk), :]
            if s > 0:
                val = val + b2_r[s - 1, :, :]
            b2_s[s, :, :] = val
            r = pltpu.make_async_remote_copy(
                src_ref=b2_s.at[s], dst_ref=b2_r.at[s],
                send_sem=b2_ss.at[s], recv_sem=b2_rs.at[s],
                device_id=(z_up,), device_id_type=pl.DeviceIdType.MESH,
            )
            r.start()
            return r

        for t in range(PLANE - 1):
            ra = a2_start(t)
            rb = b2_start(t) if t < N_PLANES - 1 else None
            ra.wait()
            if rb is not None:
                rb.wait()

        yA = a2_r[PLANE - 2, :, :] + sA[pl.ds(k * m_chunk, m_chunk), :]
        yB = b2_r[N_PLANES - 2, :, :] + sB[pl.ds(j * m_chunk, m_chunk), :]
        out_ref[:, pl.ds(0, nh)] = yA * jax.nn.sigmoid(yA)
        out_ref[:, pl.ds(nh, nh)] = yB * jax.nn.sigmoid(yB)

    f32 = jnp.float32
    return pl.pallas_call(
        body,
        out_shape=jax.ShapeDtypeStruct((m_chunk, n), f32),
        in_specs=[
            pl.BlockSpec(memory_space=pltpu.VMEM),
            pl.BlockSpec(memory_space=pltpu.VMEM),
        ],
        out_specs=pl.BlockSpec(memory_space=pltpu.VMEM),
        scratch_shapes=[
            pltpu.VMEM((m, n), f32),
            pltpu.VMEM((m, nh), f32),
            pltpu.VMEM((mp, nh), f32),
            pltpu.VMEM((mg, nh), f32),
            pltpu.VMEM((N_PLANES - 1, mp, nh), f32),
            pltpu.VMEM((N_PLANES - 1, mp, nh), f32),
            pltpu.VMEM((PLANE - 1, mg, nh), f32),
            pltpu.VMEM((PLANE - 1, mg, nh), f32),
            pltpu.VMEM((PLANE - 1, m_chunk, nh), f32),
            pltpu.VMEM((PLANE - 1, m_chunk, nh), f32),
            pltpu.VMEM((N_PLANES - 1, m_chunk, nh), f32),
            pltpu.VMEM((N_PLANES - 1, m_chunk, nh), f32),
            pltpu.SemaphoreType.DMA((N_PLANES - 1,)),
            pltpu.SemaphoreType.DMA((N_PLANES - 1,)),
            pltpu.SemaphoreType.DMA((PLANE - 1,)),
            pltpu.SemaphoreType.DMA((PLANE - 1,)),
            pltpu.SemaphoreType.DMA((PLANE - 1,)),
            pltpu.SemaphoreType.DMA((PLANE - 1,)),
            pltpu.SemaphoreType.DMA((N_PLANES - 1,)),
            pltpu.SemaphoreType.DMA((N_PLANES - 1,)),
        ],
        compiler_params=pltpu.CompilerParams(collective_id=0),
    )(x, w_mat)


# device time: 38997 ns/iter; 2.7658x vs baseline; 1.6714x over previous
import os

import jax
import jax.numpy as jnp
from jax import lax
from jax.experimental import pallas as pl
from jax.experimental.pallas import tpu as pltpu

_ABLATE = int(os.environ.get("KERNEL_ABLATE", "0"))

N_DEV = 32
NZ = 4
NY = 4
PLANE = 8
SUB = 2

X_OF = [0, 1, 1, 0, 0, 1, 1, 0]
Y_OF = [0, 0, 1, 1, 2, 2, 3, 3]
K_OF = {(X_OF[i], Y_OF[i]): i for i in range(PLANE)}
YUP = [K_OF[(X_OF[i], (Y_OF[i] + 1) % NY)] for i in range(PLANE)]
YDN = [K_OF[(X_OF[i], (Y_OF[i] + 3) % NY)] for i in range(PLANE)]


def _lut(idx, table):
    r = jnp.int32(table[0])
    for i in range(1, len(table)):
        r = jnp.where(idx == i, jnp.int32(table[i]), r)
    return r


def kernel(x, w_mat):
    m, _ = x.shape
    _, n = w_mat.shape
    mc = m // N_DEV
    nh = n // 2
    ws = nh // SUB
    mp = m // NZ
    mq = mp // 4

    def body(x_ref, w_ref, out_ref, p_ref, ystage, sa, sb, sa2, sb2,
             a1s, a1r, a2s, a2r, a3s, a3r,
             b1s, b1r, b2s, b2r, b3s, b3r,
             a1ss, a1rs, a2ss, a2rs, a3ss, a3rs,
             b1ss, b1rs, b2ss, b2rs, b3ss, b3rs):
        my = lax.axis_index("i")
        j = my // PLANE
        k = lax.rem(my, PLANE)
        y = k // 2
        xc = lax.rem(k + y, 2)
        kx = k + 1 - 2 * lax.rem(k, 2)

        z_up = lax.rem(my + PLANE, N_DEV)
        z_dn = lax.rem(my + (N_DEV - PLANE), N_DEV)
        y_up = j * PLANE + _lut(k, YUP)
        y_dn = j * PLANE + _lut(k, YDN)
        x_pt = j * PLANE + kx

        if _ABLATE != 1:
            barrier_sem = pltpu.get_barrier_semaphore()
            for nbr in (z_dn, y_dn, x_pt):
                pl.semaphore_signal(
                    barrier_sem, inc=1,
                    device_id=(nbr,), device_id_type=pl.DeviceIdType.MESH,
                )

        p_ref[:, :] = jnp.dot(
            x_ref[:, :], w_ref[:, :], preferred_element_type=jnp.float32
        )
        for yb in range(NY):
            for jp in range(NZ):
                for xp in range(2):
                    ystage[pl.ds(mp * yb + mq * jp + mc * xp, mc), :] = (
                        p_ref[pl.ds(mc * (PLANE * jp + K_OF[(xp, yb)]), mc),
                              pl.ds(nh, nh)]
                    )

        if _ABLATE == 1:
            t1 = p_ref[pl.ds(my * mc, mc), pl.ds(0, nh)]
            t2 = ystage[pl.ds(y * mc, mc), :]
            out_ref[:, pl.ds(0, nh)] = t1 * jax.nn.sigmoid(t1)
            out_ref[:, pl.ds(nh, nh)] = t2 * jax.nn.sigmoid(t2)
            return

        pl.semaphore_wait(barrier_sem, 3)

        def make_chain(n_ticks, block, target, sbuf, rbuf, ssem, rsem, h):
            rds = [None] * n_ticks

            def start(s):
                val = block(s)
                if s > 0:
                    rds[s - 1].wait()
                    val = val + rbuf[h * n_ticks + s - 1, :, :]
                sbuf[h * n_ticks + s, :, :] = val
                r = pltpu.make_async_remote_copy(
                    src_ref=sbuf.at[h * n_ticks + s],
                    dst_ref=rbuf.at[h * n_ticks + s],
                    send_sem=ssem.at[h * n_ticks + s],
                    recv_sem=rsem.at[h * n_ticks + s],
                    device_id=(target,),
                    device_id_type=pl.DeviceIdType.MESH,
                )
                r.start()
                rds[s] = r

            def finish():
                rds[n_ticks - 1].wait()
                return rbuf[h * n_ticks + n_ticks - 1, :, :]

            return start, finish

        a1 = [
            make_chain(
                NZ - 1,
                lambda s, h=h: p_ref[
                    pl.ds(lax.rem(j + (NZ - 1 - s), NZ) * mp, mp),
                    pl.ds(h * ws, ws)],
                z_up, a1s, a1r, a1ss, a1rs, h,
            )
            for h in range(SUB)
        ]
        b1 = [
            make_chain(
                NY - 1,
                lambda s, h=h: ystage[
                    pl.ds(lax.rem(y + (NY - 1 - s), NY) * mp, mp),
                    pl.ds(h * ws, ws)],
                y_up, b1s, b1r, b1ss, b1rs, h,
            )
            for h in range(SUB)
        ]
        for s in range(3):
            for h in range(SUB):
                a1[h][0](s)
                b1[h][0](s)
        for h in range(SUB):
            sa[:, pl.ds(h * ws, ws)] = (
                a1[h][1]() + p_ref[pl.ds(j * mp, mp), pl.ds(h * ws, ws)]
            )
            sb[:, pl.ds(h * ws, ws)] = (
                b1[h][1]() + ystage[pl.ds(y * mp, mp), pl.ds(h * ws, ws)]
            )

        if _ABLATE == 2:
            t1 = sa[pl.ds(k * mc, mc), :]
            t2 = sb[pl.ds(j * mc, mc), :]
            out_ref[:, pl.ds(0, nh)] = t1 * jax.nn.sigmoid(t1)
            out_ref[:, pl.ds(nh, nh)] = t2 * jax.nn.sigmoid(t2)
            return

        a2 = [
            make_chain(
                NY - 1,
                lambda s, h=h: sa[
                    pl.ds(lax.rem(y + (NY - 1 - s), NY) * mq, mq),
                    pl.ds(h * ws, ws)],
                y_up, a2s, a2r, a2ss, a2rs, h,
            )
            for h in range(SUB)
        ]
        b2 = [
            make_chain(
                NZ - 1,
                lambda s, h=h: sb[
                    pl.ds(lax.rem(j + (NZ - 1 - s), NZ) * mq, mq),
                    pl.ds(h * ws, ws)],
                z_up, b2s, b2r, b2ss, b2rs, h,
            )
            for h in range(SUB)
        ]
        for s in range(3):
            for h in range(SUB):
                a2[h][0](s)
                b2[h][0](s)
        for h in range(SUB):
            sa2[:, pl.ds(h * ws, ws)] = (
                a2[h][1]() + sa[pl.ds(y * mq, mq), pl.ds(h * ws, ws)]
            )
            sb2[:, pl.ds(h * ws, ws)] = (
                b2[h][1]() + sb[pl.ds(j * mq, mq), pl.ds(h * ws, ws)]
            )

        kpar = lax.rem(k, 2)
        a3 = [
            make_chain(
                1,
                lambda s, h=h: sa2[pl.ds((1 - kpar) * mc, mc),
                                   pl.ds(h * ws, ws)],
                x_pt, a3s, a3r, a3ss, a3rs, h,
            )
            for h in range(SUB)
        ]
        b3 = [
            make_chain(
                1,
                lambda s, h=h: sb2[pl.ds((1 - xc) * mc, mc),
                                   pl.ds(h * ws, ws)],
                x_pt, b3s, b3r, b3ss, b3rs, h,
            )
            for h in range(SUB)
        ]
        for h in range(SUB):
            a3[h][0](0)
            b3[h][0](0)
        for h in range(SUB):
            ya = a3[h][1]() + sa2[pl.ds(kpar * mc, mc), pl.ds(h * ws, ws)]
            yb = b3[h][1]() + sb2[pl.ds(xc * mc, mc), pl.ds(h * ws, ws)]
            out_ref[:, pl.ds(h * ws, ws)] = ya * jax.nn.sigmoid(ya)
            out_ref[:, pl.ds(nh + h * ws, ws)] = yb * jax.nn.sigmoid(yb)

    f32 = jnp.float32
    return pl.pallas_call(
        body,
        out_shape=jax.ShapeDtypeStruct((mc, n), f32),
        in_specs=[
            pl.BlockSpec(memory_space=pltpu.VMEM),
            pl.BlockSpec(memory_space=pltpu.VMEM),
        ],
        out_specs=pl.BlockSpec(memory_space=pltpu.VMEM),
        scratch_shapes=[
            pltpu.VMEM((m, n), f32),
            pltpu.VMEM((m, nh), f32),
            pltpu.VMEM((mp, nh), f32),
            pltpu.VMEM((mp, nh), f32),
            pltpu.VMEM((mq, nh), f32),
            pltpu.VMEM((mq, nh), f32),
            pltpu.VMEM((SUB * 3, mp, ws), f32),
            pltpu.VMEM((SUB * 3, mp, ws), f32),
            pltpu.VMEM((SUB * 3, mq, ws), f32),
            pltpu.VMEM((SUB * 3, mq, ws), f32),
            pltpu.VMEM((SUB, mc, ws), f32),
            pltpu.VMEM((SUB, mc, ws), f32),
            pltpu.VMEM((SUB * 3, mp, ws), f32),
            pltpu.VMEM((SUB * 3, mp, ws), f32),
            pltpu.VMEM((SUB * 3, mq, ws), f32),
            pltpu.VMEM((SUB * 3, mq, ws), f32),
            pltpu.VMEM((SUB, mc, ws), f32),
            pltpu.VMEM((SUB, mc, ws), f32),
            pltpu.SemaphoreType.DMA((SUB * 3,)),
            pltpu.SemaphoreType.DMA((SUB * 3,)),
            pltpu.SemaphoreType.DMA((SUB * 3,)),
            pltpu.SemaphoreType.DMA((SUB * 3,)),
            pltpu.SemaphoreType.DMA((SUB,)),
            pltpu.SemaphoreType.DMA((SUB,)),
            pltpu.SemaphoreType.DMA((SUB * 3,)),
            pltpu.SemaphoreType.DMA((SUB * 3,)),
            pltpu.SemaphoreType.DMA((SUB * 3,)),
            pltpu.SemaphoreType.DMA((SUB * 3,)),
            pltpu.SemaphoreType.DMA((SUB,)),
            pltpu.SemaphoreType.DMA((SUB,)),
        ],
        compiler_params=pltpu.CompilerParams(
            collective_id=None if _ABLATE == 1 else 0
        ),
    )(x, w_mat)
